# baseline (speedup 1.0000x reference)
import functools

import jax
import jax.numpy as jnp
from jax import lax
from jax.experimental import pallas as pl
from jax.experimental.pallas import tpu as pltpu
from jax.experimental.pallas import tpu_sc as plsc

_V = 1_000_000
_D = 64
_B = 16384
_K = 20

_NC = 2
_NS = 16
_NW = _NC * _NS

_TCH = 256
_NFULL = _V // _TCH
_TAIL_A = _NFULL * _TCH
_TAIL_N = _V - _TAIL_A
_CPW = (_NFULL + _NS - 1) // _NS

_NB = _B // _NW
_CB = 32
_NCHUNK = _NB // _CB
_IDXROW = 128
_NEG_GATHERS = _CB * _K // _IDXROW
_NVREG = _D // 16

_COMPILER_PARAMS = pltpu.CompilerParams(
    needs_layout_passes=False, use_tc_tiling_on_sc=True)


def _sc_transpose(ut, vt, ut_tail, vt_tail):
    mesh = plsc.VectorSubcoreMesh(core_axis_name="c", subcore_axis_name="s")

    @functools.partial(
        pl.kernel,
        mesh=mesh,
        compiler_params=_COMPILER_PARAMS,
        out_type=(
            jax.ShapeDtypeStruct((_V // 2, 128), jnp.float32),
            jax.ShapeDtypeStruct((_V // 2, 128), jnp.float32),
        ),
        scratch_types=[
            pltpu.VMEM((2, _D, _TCH), jnp.float32),
            pltpu.VMEM((2, _TCH // 2, 128), jnp.float32),
            pltpu.SemaphoreType.DMA,
            pltpu.SemaphoreType.DMA,
            pltpu.SemaphoreType.DMA,
            pltpu.SemaphoreType.DMA,
        ],
    )
    def tr(ut_hbm, vt_hbm, ut_tail_hbm, vt_tail_hbm, u2_hbm, v2_hbm,
           slabs, obuf, semi0, semi1, semo0, semo1):
        sid = lax.axis_index("s")
        cid = lax.axis_index("c")
        iotas = [lax.iota(jnp.int32, 16) + 16 * m for m in range(4)]
        semis = (semi0, semi1)
        semos = (semo0, semo1)

        def load_chunk(src, ci, p):
            a = ci * _TCH
            cps = []
            for g in range(8):
                cps.append(pltpu.async_copy(
                    src.at[pl.ds(8 * g, 8), pl.ds(a, _TCH)],
                    slabs.at[p, pl.ds(8 * g, 8)],
                    semis[p]))
            return cps

        def transpose_chunk(p):
            def body(q, carry):
                for half in range(2):
                    colv = jnp.full((16,), 2 * q + half, jnp.int32)
                    for m in range(4):
                        val = plsc.load_gather(
                            slabs.at[p], [iotas[m], colv])
                        obuf[p, q, pl.ds(64 * half + 16 * m, 16)] = val
                return carry
            lax.fori_loop(0, _TCH // 2, body, 0)

        def process(src, dst, tail):
            npair = (_NFULL // _NS + 1) // 2 + 1

            def pair(i, carry):
                for p in range(2):
                    ci = sid + _NS * (2 * i + p)

                    @pl.when(ci < _NFULL)
                    def _():
                        cps = load_chunk(src, ci, p)
                        for cp in cps:
                            cp.wait()
                        transpose_chunk(p)
                        pltpu.async_copy(
                            obuf.at[p],
                            dst.at[pl.ds(ci * (_TCH // 2), _TCH // 2)],
                            semos[p]).wait()
                return carry

            lax.fori_loop(0, npair, pair, 0)

            @pl.when(sid == 0)
            def _():
                pltpu.sync_copy(tail, obuf.at[0, pl.ds(0, _TAIL_N // 2)])
                pltpu.sync_copy(
                    obuf.at[0, pl.ds(0, _TAIL_N // 2)],
                    dst.at[pl.ds(_TAIL_A // 2, _TAIL_N // 2)])

        @pl.when(cid == 0)
        def _():
            process(ut_hbm, u2_hbm, ut_tail_hbm)

        @pl.when(cid == 1)
        def _():
            process(vt_hbm, v2_hbm, vt_tail_hbm)

    return tr(ut, vt, ut_tail, vt_tail)


def _sc_scores(u2, v2, cg, tg, ng, cp_, tp_, np_):
    mesh = plsc.VectorSubcoreMesh(core_axis_name="c", subcore_axis_name="s")

    @functools.partial(
        pl.kernel,
        mesh=mesh,
        compiler_params=_COMPILER_PARAMS,
        out_type=(
            jax.ShapeDtypeStruct((_B,), jnp.float32),
            jax.ShapeDtypeStruct((_B,), jnp.float32),
        ),
        scratch_types=[
            pltpu.VMEM((_NB,), jnp.int32),
            pltpu.VMEM((_NB,), jnp.int32),
            pltpu.VMEM((_NB * _K,), jnp.int32),
            pltpu.VMEM((_NB + 16,), jnp.int32),
            pltpu.VMEM((_NB + 16,), jnp.int32),
            pltpu.VMEM((_NB * _K + 16,), jnp.int32),
            pltpu.VMEM((_CB, 128), jnp.float32),
            pltpu.VMEM((_CB, 128), jnp.float32),
            pltpu.VMEM((_CB * _K, 128), jnp.float32),
            pltpu.VMEM((_NB,), jnp.float32),
            pltpu.VMEM((_NB,), jnp.float32),
            pltpu.SemaphoreType.DMA,
        ],
    )
    def scores(v2_hbm, u2_hbm, cg_hbm, tg_hbm, ng_hbm, cp_hbm, tp_hbm,
               np_hbm, pos_hbm, negs_hbm,
               cidx, tidx, nidx, cpo, tpo, npo, vrows, trows, nrows,
               posb, negb, sem):
        wid = lax.axis_index("s") * _NC + lax.axis_index("c")
        base = wid * _NB
        last_lane = lax.iota(jnp.int32, 16) == 15
        pltpu.sync_copy(cg_hbm.at[pl.ds(base, _NB)], cidx)
        pltpu.sync_copy(tg_hbm.at[pl.ds(base, _NB)], tidx)
        pltpu.sync_copy(ng_hbm.at[pl.ds(base * _K, _NB * _K)], nidx)
        pltpu.sync_copy(cp_hbm.at[pl.ds(base, _NB)], cpo.at[pl.ds(0, _NB)])
        pltpu.sync_copy(tp_hbm.at[pl.ds(base, _NB)], tpo.at[pl.ds(0, _NB)])
        pltpu.sync_copy(np_hbm.at[pl.ds(base * _K, _NB * _K)],
                        npo.at[pl.ds(0, _NB * _K)])

        for c in range(_NCHUNK):
            cb = c * _CB
            cps = [
                pltpu.async_copy(v2_hbm.at[cidx.at[pl.ds(cb, _CB)]],
                                 vrows, sem),
                pltpu.async_copy(u2_hbm.at[tidx.at[pl.ds(cb, _CB)]],
                                 trows, sem),
            ]
            for g in range(_NEG_GATHERS):
                cps.append(pltpu.async_copy(
                    u2_hbm.at[nidx.at[pl.ds(cb * _K + g * _IDXROW, _IDXROW)]],
                    nrows.at[pl.ds(g * _IDXROW, _IDXROW)],
                    sem))
            for cp in cps:
                cp.wait()

            def body(b, carry, cb=cb):
                voff = cpo[pl.ds(cb + b, 16)][0]
                toff = tpo[pl.ds(cb + b, 16)][0]
                v = [vrows[b, pl.ds(voff + 16 * j, 16)]
                     for j in range(_NVREG)]
                t = [trows[b, pl.ds(toff + 16 * j, 16)]
                     for j in range(_NVREG)]
                r0 = b * _K

                def neg_body(k, accs):
                    koff = npo[pl.ds(r0 + k, 16)][0]
                    return tuple(
                        accs[j] + nrows[r0 + k, pl.ds(koff + 16 * j, 16)]
                        for j in range(_NVREG))

                acc = lax.fori_loop(
                    0, _K, neg_body,
                    tuple(jnp.zeros((16,), jnp.float32)
                          for _ in range(_NVREG)))
                pos_l = t[0] * v[0]
                neg_l = acc[0] * v[0]
                for j in range(1, _NVREG):
                    pos_l = pos_l + t[j] * v[j]
                    neg_l = neg_l + acc[j] * v[j]
                out_idx = jnp.full((16,), cb + b, jnp.int32)
                plsc.store_scatter(posb, [out_idx], plsc.cumsum(pos_l),
                                   mask=last_lane)
                plsc.store_scatter(negb, [out_idx], plsc.cumsum(neg_l),
                                   mask=last_lane)
                return carry

            lax.fori_loop(0, _CB, body, 0)

        pltpu.sync_copy(posb, pos_hbm.at[pl.ds(base, _NB)])
        pltpu.sync_copy(negb, negs_hbm.at[pl.ds(base, _NB)])

    return scores(v2, u2, cg, tg, ng, cp_, tp_, np_)


def _loss_tc(pos, neg):
    def body(pos_ref, neg_ref, out_ref):
        loss = (jax.nn.log_sigmoid(pos_ref[...])
                + jax.nn.log_sigmoid(-neg_ref[...]))
        out_ref[0, 0] = -jnp.sum(loss) / _B

    return pl.pallas_call(
        body,
        out_shape=jax.ShapeDtypeStruct((1, 1), jnp.float32),
        out_specs=pl.BlockSpec(memory_space=pltpu.SMEM),
    )(pos.reshape(128, 128), neg.reshape(128, 128))


def kernel(center_words, target_words, negative_words, embedding_v, embedding_u):
    c = center_words.reshape(-1).astype(jnp.int32)
    t = target_words.reshape(-1).astype(jnp.int32)
    n = negative_words.reshape(-1).astype(jnp.int32)
    u_tail = embedding_u[_TAIL_A:, :].reshape(_TAIL_N // 2, 128)
    v_tail = embedding_v[_TAIL_A:, :].reshape(_TAIL_N // 2, 128)
    u2, v2 = _sc_transpose(embedding_u.T, embedding_v.T, u_tail, v_tail)
    pos, neg = _sc_scores(
        u2, v2,
        c >> 1, t >> 1, n >> 1,
        (c & 1) * 64, (t & 1) * 64, (n & 1) * 64,
    )
    return _loss_tc(pos, neg)[0, 0]


# speedup vs baseline: 1.1638x; 1.1638x over previous
import functools

import jax
import jax.numpy as jnp
from jax import lax
from jax.experimental import pallas as pl
from jax.experimental.pallas import tpu as pltpu
from jax.experimental.pallas import tpu_sc as plsc

_V = 1_000_000
_D = 64
_B = 16384
_K = 20

_NC = 2
_NS = 16
_NW = _NC * _NS

_TCH = 256
_NFULL = _V // _TCH
_TAIL_A = _NFULL * _TCH
_TAIL_N = _V - _TAIL_A
_CPW = (_NFULL + _NS - 1) // _NS

_NB = _B // _NW
_CB = 32
_NCHUNK = _NB // _CB
_IDXROW = 128
_NEG_GATHERS = _CB * _K // _IDXROW
_NVREG = _D // 16

_COMPILER_PARAMS = pltpu.CompilerParams(
    needs_layout_passes=False, use_tc_tiling_on_sc=True)


def _sc_transpose(ut, vt, ut_tail, vt_tail):
    mesh = plsc.VectorSubcoreMesh(core_axis_name="c", subcore_axis_name="s")

    @functools.partial(
        pl.kernel,
        mesh=mesh,
        compiler_params=_COMPILER_PARAMS,
        out_type=(
            jax.ShapeDtypeStruct((_V // 2, 128), jnp.float32),
            jax.ShapeDtypeStruct((_V // 2, 128), jnp.float32),
        ),
        scratch_types=[
            pltpu.VMEM((2, _D, _TCH), jnp.float32),
            pltpu.VMEM((2, _TCH // 2, 128), jnp.float32),
            pltpu.SemaphoreType.DMA,
            pltpu.SemaphoreType.DMA,
            pltpu.SemaphoreType.DMA,
            pltpu.SemaphoreType.DMA,
        ],
    )
    def tr(ut_hbm, vt_hbm, ut_tail_hbm, vt_tail_hbm, u2_hbm, v2_hbm,
           slabs, obuf, semi0, semi1, semo0, semo1):
        sid = lax.axis_index("s")
        cid = lax.axis_index("c")
        iotas = [lax.iota(jnp.int32, 16) + 16 * m for m in range(4)]
        semis = (semi0, semi1)
        semos = (semo0, semo1)

        def load_chunk(src, ci, p):
            a = ci * _TCH
            cps = []
            for g in range(8):
                cps.append(pltpu.async_copy(
                    src.at[pl.ds(8 * g, 8), pl.ds(a, _TCH)],
                    slabs.at[p, pl.ds(8 * g, 8)],
                    semis[p]))
            return cps

        def transpose_chunk(p):
            def body(q, carry):
                for half in range(2):
                    colv = jnp.full((16,), 2 * q + half, jnp.int32)
                    for m in range(4):
                        val = plsc.load_gather(
                            slabs.at[p], [iotas[m], colv])
                        obuf[p, q, pl.ds(64 * half + 16 * m, 16)] = val
                return carry
            lax.fori_loop(0, _TCH // 2, body, 0)

        def process(src, dst, tail):
            npair = (_NFULL // _NS + 1) // 2 + 1

            load_chunk(src, sid, 0)
            load_chunk(src, sid + _NS, 1)

            def pair(i, carry):
                for p in range(2):
                    ci = sid + _NS * (2 * i + p)

                    @pl.when(ci < _NFULL)
                    def _(p=p, ci=ci):
                        for g in range(8):
                            pltpu.make_async_copy(
                                src.at[pl.ds(8 * g, 8), pl.ds(ci * _TCH, _TCH)],
                                slabs.at[p, pl.ds(8 * g, 8)],
                                semis[p]).wait()
                        @pl.when(i > 0)
                        def _():
                            pltpu.make_async_copy(
                                obuf.at[p],
                                dst.at[pl.ds(0, _TCH // 2)],
                                semos[p]).wait()
                        transpose_chunk(p)
                        cin = ci + 2 * _NS

                        @pl.when(cin < _NFULL)
                        def _():
                            load_chunk(src, cin, p)

                        pltpu.async_copy(
                            obuf.at[p],
                            dst.at[pl.ds(ci * (_TCH // 2), _TCH // 2)],
                            semos[p])
                return carry

            lax.fori_loop(0, npair, pair, 0)
            for p in range(2):
                pltpu.make_async_copy(
                    obuf.at[p], dst.at[pl.ds(0, _TCH // 2)],
                    semos[p]).wait()

            @pl.when(sid == 0)
            def _():
                pltpu.sync_copy(tail, obuf.at[0, pl.ds(0, _TAIL_N // 2)])
                pltpu.sync_copy(
                    obuf.at[0, pl.ds(0, _TAIL_N // 2)],
                    dst.at[pl.ds(_TAIL_A // 2, _TAIL_N // 2)])

        @pl.when(cid == 0)
        def _():
            process(ut_hbm, u2_hbm, ut_tail_hbm)

        @pl.when(cid == 1)
        def _():
            process(vt_hbm, v2_hbm, vt_tail_hbm)

    return tr(ut, vt, ut_tail, vt_tail)


def _sc_scores(u2, v2, cg, tg, ng, cp_, tp_, np_):
    mesh = plsc.VectorSubcoreMesh(core_axis_name="c", subcore_axis_name="s")

    @functools.partial(
        pl.kernel,
        mesh=mesh,
        compiler_params=_COMPILER_PARAMS,
        out_type=(
            jax.ShapeDtypeStruct((_B,), jnp.float32),
            jax.ShapeDtypeStruct((_B,), jnp.float32),
        ),
        scratch_types=[
            pltpu.VMEM((_NB,), jnp.int32),
            pltpu.VMEM((_NB,), jnp.int32),
            pltpu.VMEM((_NB * _K,), jnp.int32),
            pltpu.VMEM((_NB + 16,), jnp.int32),
            pltpu.VMEM((_NB + 16,), jnp.int32),
            pltpu.VMEM((_NB * _K + 16,), jnp.int32),
            pltpu.VMEM((_CB, 128), jnp.float32),
            pltpu.VMEM((_CB, 128), jnp.float32),
            pltpu.VMEM((_CB * _K, 128), jnp.float32),
            pltpu.VMEM((_NB,), jnp.float32),
            pltpu.VMEM((_NB,), jnp.float32),
            pltpu.SemaphoreType.DMA,
        ],
    )
    def scores(v2_hbm, u2_hbm, cg_hbm, tg_hbm, ng_hbm, cp_hbm, tp_hbm,
               np_hbm, pos_hbm, negs_hbm,
               cidx, tidx, nidx, cpo, tpo, npo, vrows, trows, nrows,
               posb, negb, sem):
        wid = lax.axis_index("s") * _NC + lax.axis_index("c")
        base = wid * _NB
        last_lane = lax.iota(jnp.int32, 16) == 15
        pltpu.sync_copy(cg_hbm.at[pl.ds(base, _NB)], cidx)
        pltpu.sync_copy(tg_hbm.at[pl.ds(base, _NB)], tidx)
        pltpu.sync_copy(ng_hbm.at[pl.ds(base * _K, _NB * _K)], nidx)
        pltpu.sync_copy(cp_hbm.at[pl.ds(base, _NB)], cpo.at[pl.ds(0, _NB)])
        pltpu.sync_copy(tp_hbm.at[pl.ds(base, _NB)], tpo.at[pl.ds(0, _NB)])
        pltpu.sync_copy(np_hbm.at[pl.ds(base * _K, _NB * _K)],
                        npo.at[pl.ds(0, _NB * _K)])

        for c in range(_NCHUNK):
            cb = c * _CB
            cps = [
                pltpu.async_copy(v2_hbm.at[cidx.at[pl.ds(cb, _CB)]],
                                 vrows, sem),
                pltpu.async_copy(u2_hbm.at[tidx.at[pl.ds(cb, _CB)]],
                                 trows, sem),
            ]
            for g in range(_NEG_GATHERS):
                cps.append(pltpu.async_copy(
                    u2_hbm.at[nidx.at[pl.ds(cb * _K + g * _IDXROW, _IDXROW)]],
                    nrows.at[pl.ds(g * _IDXROW, _IDXROW)],
                    sem))
            for cp in cps:
                cp.wait()

            def body(b, carry, cb=cb):
                voff = cpo[pl.ds(cb + b, 16)][0]
                toff = tpo[pl.ds(cb + b, 16)][0]
                v = [vrows[b, pl.ds(voff + 16 * j, 16)]
                     for j in range(_NVREG)]
                t = [trows[b, pl.ds(toff + 16 * j, 16)]
                     for j in range(_NVREG)]
                r0 = b * _K

                def neg_body(k, accs):
                    koff = npo[pl.ds(r0 + k, 16)][0]
                    return tuple(
                        accs[j] + nrows[r0 + k, pl.ds(koff + 16 * j, 16)]
                        for j in range(_NVREG))

                acc = lax.fori_loop(
                    0, _K, neg_body,
                    tuple(jnp.zeros((16,), jnp.float32)
                          for _ in range(_NVREG)))
                pos_l = t[0] * v[0]
                neg_l = acc[0] * v[0]
                for j in range(1, _NVREG):
                    pos_l = pos_l + t[j] * v[j]
                    neg_l = neg_l + acc[j] * v[j]
                out_idx = jnp.full((16,), cb + b, jnp.int32)
                plsc.store_scatter(posb, [out_idx], plsc.cumsum(pos_l),
                                   mask=last_lane)
                plsc.store_scatter(negb, [out_idx], plsc.cumsum(neg_l),
                                   mask=last_lane)
                return carry

            lax.fori_loop(0, _CB, body, 0)

        pltpu.sync_copy(posb, pos_hbm.at[pl.ds(base, _NB)])
        pltpu.sync_copy(negb, negs_hbm.at[pl.ds(base, _NB)])

    return scores(v2, u2, cg, tg, ng, cp_, tp_, np_)


def _loss_tc(pos, neg):
    def body(pos_ref, neg_ref, out_ref):
        loss = (jax.nn.log_sigmoid(pos_ref[...])
                + jax.nn.log_sigmoid(-neg_ref[...]))
        out_ref[0, 0] = -jnp.sum(loss) / _B

    return pl.pallas_call(
        body,
        out_shape=jax.ShapeDtypeStruct((1, 1), jnp.float32),
        out_specs=pl.BlockSpec(memory_space=pltpu.SMEM),
    )(pos.reshape(128, 128), neg.reshape(128, 128))


def kernel(center_words, target_words, negative_words, embedding_v, embedding_u):
    c = center_words.reshape(-1).astype(jnp.int32)
    t = target_words.reshape(-1).astype(jnp.int32)
    n = negative_words.reshape(-1).astype(jnp.int32)
    u_tail = embedding_u[_TAIL_A:, :].reshape(_TAIL_N // 2, 128)
    v_tail = embedding_v[_TAIL_A:, :].reshape(_TAIL_N // 2, 128)
    u2, v2 = _sc_transpose(embedding_u.T, embedding_v.T, u_tail, v_tail)
    pos, neg = _sc_scores(
        u2, v2,
        c >> 1, t >> 1, n >> 1,
        (c & 1) * 64, (t & 1) * 64, (n & 1) * 64,
    )
    return _loss_tc(pos, neg)[0, 0]


# speedup vs baseline: 1.1662x; 1.0021x over previous
import functools

import jax
import jax.numpy as jnp
from jax import lax
from jax.experimental import pallas as pl
from jax.experimental.pallas import tpu as pltpu
from jax.experimental.pallas import tpu_sc as plsc

_V = 1_000_000
_D = 64
_B = 16384
_K = 20

_NC = 2
_NS = 16
_NW = _NC * _NS

_TCH = 256
_NFULL = _V // _TCH
_TAIL_A = _NFULL * _TCH
_TAIL_N = _V - _TAIL_A
_CPW = (_NFULL + _NS - 1) // _NS

_NB = _B // _NW
_CB = 32
_NCHUNK = _NB // _CB
_IDXROW = 128
_NEG_GATHERS = _CB * _K // _IDXROW
_NVREG = _D // 16

_COMPILER_PARAMS = pltpu.CompilerParams(
    needs_layout_passes=False, use_tc_tiling_on_sc=True)


def _sc_transpose(ut, vt, ut_tail, vt_tail):
    mesh = plsc.VectorSubcoreMesh(core_axis_name="c", subcore_axis_name="s")

    @functools.partial(
        pl.kernel,
        mesh=mesh,
        compiler_params=_COMPILER_PARAMS,
        out_type=(
            jax.ShapeDtypeStruct((_V // 2, 128), jnp.float32),
            jax.ShapeDtypeStruct((_V // 2, 128), jnp.float32),
        ),
        scratch_types=[
            pltpu.VMEM((2, _D, _TCH + 1), jnp.float32),
            pltpu.VMEM((2, _TCH // 2, 128), jnp.float32),
            pltpu.SemaphoreType.DMA,
            pltpu.SemaphoreType.DMA,
            pltpu.SemaphoreType.DMA,
            pltpu.SemaphoreType.DMA,
        ],
    )
    def tr(ut_hbm, vt_hbm, ut_tail_hbm, vt_tail_hbm, u2_hbm, v2_hbm,
           slabs, obuf, semi0, semi1, semo0, semo1):
        sid = lax.axis_index("s")
        cid = lax.axis_index("c")
        iotas = [lax.iota(jnp.int32, 16) + 16 * m for m in range(4)]
        semis = (semi0, semi1)
        semos = (semo0, semo1)

        def load_chunk(src, ci, p):
            return pltpu.async_copy(
                src.at[pl.ds(0, _D), pl.ds(ci * _TCH, _TCH)],
                slabs.at[p, pl.ds(0, _D), pl.ds(0, _TCH)],
                semis[p])

        def transpose_chunk(p):
            def body(q, carry):
                for half in range(2):
                    colv = jnp.full((16,), 2 * q + half, jnp.int32)
                    for m in range(4):
                        val = plsc.load_gather(
                            slabs.at[p], [iotas[m], colv])
                        obuf[p, q, pl.ds(64 * half + 16 * m, 16)] = val
                return carry
            lax.fori_loop(0, _TCH // 2, body, 0)

        def process(src, dst, tail):
            npair = (_NFULL // _NS + 1) // 2 + 1

            load_chunk(src, sid, 0)
            load_chunk(src, sid + _NS, 1)

            def pair(i, carry):
                for p in range(2):
                    ci = sid + _NS * (2 * i + p)

                    @pl.when(ci < _NFULL)
                    def _(p=p, ci=ci):
                        pltpu.make_async_copy(
                            src.at[pl.ds(0, _D), pl.ds(ci * _TCH, _TCH)],
                            slabs.at[p, pl.ds(0, _D), pl.ds(0, _TCH)],
                            semis[p]).wait()
                        @pl.when(i > 0)
                        def _():
                            pltpu.make_async_copy(
                                obuf.at[p],
                                dst.at[pl.ds(0, _TCH // 2)],
                                semos[p]).wait()
                        transpose_chunk(p)
                        cin = ci + 2 * _NS

                        @pl.when(cin < _NFULL)
                        def _():
                            load_chunk(src, cin, p)

                        pltpu.async_copy(
                            obuf.at[p],
                            dst.at[pl.ds(ci * (_TCH // 2), _TCH // 2)],
                            semos[p])
                return carry

            lax.fori_loop(0, npair, pair, 0)
            for p in range(2):
                pltpu.make_async_copy(
                    obuf.at[p], dst.at[pl.ds(0, _TCH // 2)],
                    semos[p]).wait()

            @pl.when(sid == 0)
            def _():
                pltpu.sync_copy(tail, obuf.at[0, pl.ds(0, _TAIL_N // 2)])
                pltpu.sync_copy(
                    obuf.at[0, pl.ds(0, _TAIL_N // 2)],
                    dst.at[pl.ds(_TAIL_A // 2, _TAIL_N // 2)])

        @pl.when(cid == 0)
        def _():
            process(ut_hbm, u2_hbm, ut_tail_hbm)

        @pl.when(cid == 1)
        def _():
            process(vt_hbm, v2_hbm, vt_tail_hbm)

    return tr(ut, vt, ut_tail, vt_tail)


def _sc_scores(u2, v2, cg, tg, ng, cp_, tp_, np_):
    mesh = plsc.VectorSubcoreMesh(core_axis_name="c", subcore_axis_name="s")

    @functools.partial(
        pl.kernel,
        mesh=mesh,
        compiler_params=_COMPILER_PARAMS,
        out_type=(
            jax.ShapeDtypeStruct((_B,), jnp.float32),
            jax.ShapeDtypeStruct((_B,), jnp.float32),
        ),
        scratch_types=[
            pltpu.VMEM((_NB,), jnp.int32),
            pltpu.VMEM((_NB,), jnp.int32),
            pltpu.VMEM((_NB * _K,), jnp.int32),
            pltpu.VMEM((_NB + 16,), jnp.int32),
            pltpu.VMEM((_NB + 16,), jnp.int32),
            pltpu.VMEM((_NB * _K + 16,), jnp.int32),
            pltpu.VMEM((_CB, 128), jnp.float32),
            pltpu.VMEM((_CB, 128), jnp.float32),
            pltpu.VMEM((_CB * _K, 128), jnp.float32),
            pltpu.VMEM((_NB,), jnp.float32),
            pltpu.VMEM((_NB,), jnp.float32),
            pltpu.SemaphoreType.DMA,
        ],
    )
    def scores(v2_hbm, u2_hbm, cg_hbm, tg_hbm, ng_hbm, cp_hbm, tp_hbm,
               np_hbm, pos_hbm, negs_hbm,
               cidx, tidx, nidx, cpo, tpo, npo, vrows, trows, nrows,
               posb, negb, sem):
        wid = lax.axis_index("s") * _NC + lax.axis_index("c")
        base = wid * _NB
        last_lane = lax.iota(jnp.int32, 16) == 15
        pltpu.sync_copy(cg_hbm.at[pl.ds(base, _NB)], cidx)
        pltpu.sync_copy(tg_hbm.at[pl.ds(base, _NB)], tidx)
        pltpu.sync_copy(ng_hbm.at[pl.ds(base * _K, _NB * _K)], nidx)
        pltpu.sync_copy(cp_hbm.at[pl.ds(base, _NB)], cpo.at[pl.ds(0, _NB)])
        pltpu.sync_copy(tp_hbm.at[pl.ds(base, _NB)], tpo.at[pl.ds(0, _NB)])
        pltpu.sync_copy(np_hbm.at[pl.ds(base * _K, _NB * _K)],
                        npo.at[pl.ds(0, _NB * _K)])

        for c in range(_NCHUNK):
            cb = c * _CB
            cps = [
                pltpu.async_copy(v2_hbm.at[cidx.at[pl.ds(cb, _CB)]],
                                 vrows, sem),
                pltpu.async_copy(u2_hbm.at[tidx.at[pl.ds(cb, _CB)]],
                                 trows, sem),
            ]
            for g in range(_NEG_GATHERS):
                cps.append(pltpu.async_copy(
                    u2_hbm.at[nidx.at[pl.ds(cb * _K + g * _IDXROW, _IDXROW)]],
                    nrows.at[pl.ds(g * _IDXROW, _IDXROW)],
                    sem))
            for cp in cps:
                cp.wait()

            def body(b, carry, cb=cb):
                voff = cpo[pl.ds(cb + b, 16)][0]
                toff = tpo[pl.ds(cb + b, 16)][0]
                v = [vrows[b, pl.ds(voff + 16 * j, 16)]
                     for j in range(_NVREG)]
                t = [trows[b, pl.ds(toff + 16 * j, 16)]
                     for j in range(_NVREG)]
                r0 = b * _K

                def neg_body(k, accs):
                    koff = npo[pl.ds(r0 + k, 16)][0]
                    return tuple(
                        accs[j] + nrows[r0 + k, pl.ds(koff + 16 * j, 16)]
                        for j in range(_NVREG))

                acc = lax.fori_loop(
                    0, _K, neg_body,
                    tuple(jnp.zeros((16,), jnp.float32)
                          for _ in range(_NVREG)))
                pos_l = t[0] * v[0]
                neg_l = acc[0] * v[0]
                for j in range(1, _NVREG):
                    pos_l = pos_l + t[j] * v[j]
                    neg_l = neg_l + acc[j] * v[j]
                out_idx = jnp.full((16,), cb + b, jnp.int32)
                plsc.store_scatter(posb, [out_idx], plsc.cumsum(pos_l),
                                   mask=last_lane)
                plsc.store_scatter(negb, [out_idx], plsc.cumsum(neg_l),
                                   mask=last_lane)
                return carry

            lax.fori_loop(0, _CB, body, 0)

        pltpu.sync_copy(posb, pos_hbm.at[pl.ds(base, _NB)])
        pltpu.sync_copy(negb, negs_hbm.at[pl.ds(base, _NB)])

    return scores(v2, u2, cg, tg, ng, cp_, tp_, np_)


def _loss_tc(pos, neg):
    def body(pos_ref, neg_ref, out_ref):
        loss = (jax.nn.log_sigmoid(pos_ref[...])
                + jax.nn.log_sigmoid(-neg_ref[...]))
        out_ref[0, 0] = -jnp.sum(loss) / _B

    return pl.pallas_call(
        body,
        out_shape=jax.ShapeDtypeStruct((1, 1), jnp.float32),
        out_specs=pl.BlockSpec(memory_space=pltpu.SMEM),
    )(pos.reshape(128, 128), neg.reshape(128, 128))


def kernel(center_words, target_words, negative_words, embedding_v, embedding_u):
    c = center_words.reshape(-1).astype(jnp.int32)
    t = target_words.reshape(-1).astype(jnp.int32)
    n = negative_words.reshape(-1).astype(jnp.int32)
    u_tail = embedding_u[_TAIL_A:, :].reshape(_TAIL_N // 2, 128)
    v_tail = embedding_v[_TAIL_A:, :].reshape(_TAIL_N // 2, 128)
    u2, v2 = _sc_transpose(embedding_u.T, embedding_v.T, u_tail, v_tail)
    pos, neg = _sc_scores(
        u2, v2,
        c >> 1, t >> 1, n >> 1,
        (c & 1) * 64, (t & 1) * 64, (n & 1) * 64,
    )
    return _loss_tc(pos, neg)[0, 0]
